# Initial kernel scaffold; baseline (speedup 1.0000x reference)
import jax
import jax.numpy as jnp
from jax.experimental import pallas as pl


def kernel(x, edge_index, kernel_idx, W1, gamma1, beta1, W2, gamma2, beta2):
    raise NotImplementedError("write your pallas kernel here")


# speedup vs baseline: 3.2886x; 3.2886x over previous
import functools

import jax
import jax.numpy as jnp
from jax import lax
from jax.experimental import pallas as pl
from jax.experimental.pallas import tpu as pltpu
from jax.experimental.pallas import tpu_sc as plsc

_EPS = 1e-5
_SLOPE = 0.01


def _mm_body(x_ref, w_ref, o_ref):
    o_ref[0] = jnp.dot(x_ref[...], w_ref[0], preferred_element_type=jnp.float32)


def _xk(x, W, tn=2000):
    n, c = x.shape
    k = W.shape[0]
    return pl.pallas_call(
        _mm_body,
        grid=(n // tn, k),
        in_specs=[
            pl.BlockSpec((tn, c), lambda i, j: (i, 0)),
            pl.BlockSpec((1, c, c), lambda i, j: (j, 0, 0)),
        ],
        out_specs=pl.BlockSpec((1, tn, c), lambda i, j: (j, i, 0)),
        out_shape=jax.ShapeDtypeStruct((k, n, c), jnp.float32),
    )(x, W)


def _fidx_body(n, k_ref, s_ref, o_ref):
    o_ref[...] = k_ref[...] * n + s_ref[...]


def _fidx(kidx, src, n):
    e = kidx.shape[0]
    k2 = kidx.reshape(e // 128, 128)
    s2 = src.reshape(e // 128, 128)
    out = pl.pallas_call(
        functools.partial(_fidx_body, n),
        out_shape=jax.ShapeDtypeStruct(k2.shape, jnp.int32),
    )(k2, s2)
    return out.reshape(e)


def _sc_conv(xk_flat, fidx3, dst3, zeros, n, c, nblk):
    nwb, blk, ch = fidx3.shape
    n_cores, n_sub = 2, 16
    rps = (n // n_sub) // 8 * 8
    tail = n - rps * n_sub
    mesh = plsc.VectorSubcoreMesh(core_axis_name="c", subcore_axis_name="s")

    @functools.partial(
        pl.kernel,
        out_type=jax.ShapeDtypeStruct((n_cores, n, c), jnp.float32),
        mesh=mesh,
        scratch_types=[
            pltpu.VMEM((blk, ch), jnp.int32),
            pltpu.VMEM((blk, ch), jnp.int32),
            pltpu.VMEM((ch, c), jnp.float32),
            pltpu.VMEM((ch, c), jnp.float32),
            pltpu.VMEM_SHARED((n, c), jnp.float32),
            pltpu.SemaphoreType.DMA,
            pltpu.SemaphoreType.DMA,
        ],
    )
    def conv(xk_hbm, fidx_hbm, dst_hbm, z_hbm, out_hbm,
             fidx_v, dst_v, rows0, rows1, accum, gsem, ssem):
        cid = lax.axis_index("c")
        sid = lax.axis_index("s")
        wid = cid * n_sub + sid
        rows = (rows0, rows1)

        r0 = sid * rps
        pltpu.sync_copy(z_hbm.at[pl.ds(r0, rps)], accum.at[pl.ds(r0, rps)])
        if tail:
            @pl.when(sid == 0)
            def _():
                pltpu.sync_copy(z_hbm.at[pl.ds(rps * n_sub, tail)],
                                accum.at[pl.ds(rps * n_sub, tail)])
        plsc.subcore_barrier()

        def g_start(i, b):
            return pltpu.async_copy(xk_hbm.at[fidx_v.at[i]], rows[b], gsem)

        def s_start(i, b):
            return pltpu.async_copy(rows[b], accum.at[dst_v.at[i]],
                                    ssem, add=True)

        @pl.loop(0, nblk)
        def _(b):
            plane = wid * nblk + b
            pltpu.sync_copy(fidx_hbm.at[plane], fidx_v)
            pltpu.sync_copy(dst_hbm.at[plane], dst_v)

            @pl.loop(0, blk // 2)
            def _(p):
                c0 = 2 * p
                hg0 = g_start(c0, 0)
                hg1 = g_start(c0 + 1, 1)
                hg0.wait()
                hg1.wait()
                hs0 = s_start(c0, 0)
                hs1 = s_start(c0 + 1, 1)
                hs0.wait()
                hs1.wait()
            if blk % 2:
                hg = g_start(blk - 1, 0)
                hg.wait()
                s_start(blk - 1, 0).wait()

        plsc.subcore_barrier()
        pltpu.sync_copy(accum.at[pl.ds(r0, rps)],
                        out_hbm.at[cid, pl.ds(r0, rps)])
        if tail:
            @pl.when(sid == 0)
            def _():
                pltpu.sync_copy(accum.at[pl.ds(rps * n_sub, tail)],
                                out_hbm.at[cid, pl.ds(rps * n_sub, tail)])

    return conv(xk_flat, fidx3, dst3, zeros)


def _norm_mm_body(tn, p_ref, g_ref, b_ref, w_ref, o_ref, h_ref):
    @pl.when((pl.program_id(0) == 0) & (pl.program_id(1) == 0))
    def _():
        h = p_ref[0] + p_ref[1]
        mu = jnp.mean(h, axis=0, keepdims=True)
        d = h - mu
        var = jnp.mean(d * d, axis=0, keepdims=True)
        y = d * lax.rsqrt(var + _EPS) * g_ref[...] + b_ref[...]
        h_ref[...] = jnp.where(y >= 0, y, _SLOPE * y)

    i = pl.program_id(0)
    o_ref[0] = jnp.dot(h_ref[pl.ds(i * tn, tn), :], w_ref[0],
                       preferred_element_type=jnp.float32)


def _norm_mm(p, gamma, beta, W, tn=2000):
    n, c = p.shape[1], p.shape[2]
    k = W.shape[0]
    return pl.pallas_call(
        functools.partial(_norm_mm_body, tn),
        grid=(n // tn, k),
        in_specs=[
            pl.BlockSpec((2, n, c), lambda i, j: (0, 0, 0)),
            pl.BlockSpec((1, c), lambda i, j: (0, 0)),
            pl.BlockSpec((1, c), lambda i, j: (0, 0)),
            pl.BlockSpec((1, c, c), lambda i, j: (j, 0, 0)),
        ],
        out_specs=pl.BlockSpec((1, tn, c), lambda i, j: (j, i, 0)),
        out_shape=jax.ShapeDtypeStruct((k, n, c), jnp.float32),
        scratch_shapes=[pltpu.VMEM((n, c), jnp.float32)],
    )(p, gamma.reshape(1, c), beta.reshape(1, c), W)


def _norm2_body(p_ref, g_ref, b_ref, r_ref, o_ref):
    h = p_ref[0] + p_ref[1]
    mu = jnp.mean(h, axis=0, keepdims=True)
    d = h - mu
    var = jnp.mean(d * d, axis=0, keepdims=True)
    y = d * lax.rsqrt(var + _EPS) * g_ref[...] + b_ref[...] + r_ref[...]
    o_ref[...] = jnp.where(y >= 0, y, _SLOPE * y)


def _norm2(p, gamma, beta, resid):
    n, c = p.shape[1], p.shape[2]
    return pl.pallas_call(
        _norm2_body,
        out_shape=jax.ShapeDtypeStruct((n, c), jnp.float32),
    )(p, gamma.reshape(1, c), beta.reshape(1, c), resid)


def kernel(x, edge_index, kernel_idx, W1, gamma1, beta1, W2, gamma2, beta2):
    n, c = x.shape
    k = W1.shape[0]
    src = edge_index[0]
    dst = edge_index[1]
    e = src.shape[0]
    nw, ch, nblk = 32, 80, 5
    blk = e // (nw * ch * nblk)
    fidx3 = _fidx(kernel_idx, src, n).reshape(nw * nblk, blk, ch)
    dst3 = dst.reshape(nw * nblk, blk, ch)
    zeros = jnp.zeros((n, c), jnp.float32)

    xk1 = _xk(x, W1)
    p1 = _sc_conv(xk1.reshape(k * n, c), fidx3, dst3, zeros, n, c, nblk)
    xk2 = _norm_mm(p1, gamma1, beta1, W2)
    p2 = _sc_conv(xk2.reshape(k * n, c), fidx3, dst3, zeros, n, c, nblk)
    return _norm2(p2, gamma2, beta2, x)


# speedup vs baseline: 3.4551x; 1.0506x over previous
import functools

import jax
import jax.numpy as jnp
from jax import lax
from jax.experimental import pallas as pl
from jax.experimental.pallas import tpu as pltpu
from jax.experimental.pallas import tpu_sc as plsc

_EPS = 1e-5
_SLOPE = 0.01


def _mm_body(x_ref, w_ref, o_ref):
    o_ref[0] = jnp.dot(x_ref[...], w_ref[0], preferred_element_type=jnp.float32)


def _xk(x, W, tn=2000):
    n, c = x.shape
    k = W.shape[0]
    return pl.pallas_call(
        _mm_body,
        grid=(n // tn, k),
        in_specs=[
            pl.BlockSpec((tn, c), lambda i, j: (i, 0)),
            pl.BlockSpec((1, c, c), lambda i, j: (j, 0, 0)),
        ],
        out_specs=pl.BlockSpec((1, tn, c), lambda i, j: (j, i, 0)),
        out_shape=jax.ShapeDtypeStruct((k, n, c), jnp.float32),
    )(x, W)


def _fidx_body(n, k_ref, s_ref, o_ref):
    o_ref[...] = k_ref[...] * n + s_ref[...]


def _fidx(kidx, src, n):
    e = kidx.shape[0]
    k2 = kidx.reshape(e // 128, 128)
    s2 = src.reshape(e // 128, 128)
    out = pl.pallas_call(
        functools.partial(_fidx_body, n),
        out_shape=jax.ShapeDtypeStruct(k2.shape, jnp.int32),
    )(k2, s2)
    return out.reshape(e)


def _sc_conv(xk_flat, fidx3, dst3, zeros, n, c, nblk):
    nwb, blk, ch = fidx3.shape
    n_cores, n_sub = 2, 16
    rps = (n // n_sub) // 8 * 8
    tail = n - rps * n_sub
    mesh = plsc.VectorSubcoreMesh(core_axis_name="c", subcore_axis_name="s")

    @functools.partial(
        pl.kernel,
        out_type=jax.ShapeDtypeStruct((n_cores, n, c), jnp.float32),
        mesh=mesh,
        scratch_types=[
            pltpu.VMEM((blk, ch), jnp.int32),
            pltpu.VMEM((blk, ch), jnp.int32),
            pltpu.VMEM((ch, c), jnp.float32),
            pltpu.VMEM((ch, c), jnp.float32),
            pltpu.VMEM_SHARED((n, c), jnp.float32),
            pltpu.SemaphoreType.DMA,
            pltpu.SemaphoreType.DMA,
        ],
    )
    def conv(xk_hbm, fidx_hbm, dst_hbm, z_hbm, out_hbm,
             fidx_v, dst_v, rows0, rows1, accum, gsem, ssem):
        cid = lax.axis_index("c")
        sid = lax.axis_index("s")
        wid = cid * n_sub + sid
        rows = (rows0, rows1)

        r0 = sid * rps
        pltpu.sync_copy(z_hbm.at[pl.ds(r0, rps)], accum.at[pl.ds(r0, rps)])
        if tail:
            @pl.when(sid == 0)
            def _():
                pltpu.sync_copy(z_hbm.at[pl.ds(rps * n_sub, tail)],
                                accum.at[pl.ds(rps * n_sub, tail)])
        plsc.subcore_barrier()

        def g_start(i, b):
            return pltpu.async_copy(xk_hbm.at[fidx_v.at[i]], rows[b], gsem)

        def s_start(i, b):
            return pltpu.async_copy(rows[b], accum.at[dst_v.at[i]],
                                    ssem, add=True)

        @pl.loop(0, nblk)
        def _(b):
            plane = wid * nblk + b
            pltpu.sync_copy(fidx_hbm.at[plane], fidx_v)
            pltpu.sync_copy(dst_hbm.at[plane], dst_v)

            @pl.loop(0, blk // 2)
            def _(p):
                c0 = 2 * p
                hg0 = g_start(c0, 0)
                hg1 = g_start(c0 + 1, 1)
                hg0.wait()
                hg1.wait()
                hs0 = s_start(c0, 0)
                hs1 = s_start(c0 + 1, 1)
                hs0.wait()
                hs1.wait()
            if blk % 2:
                hg = g_start(blk - 1, 0)
                hg.wait()
                s_start(blk - 1, 0).wait()

        plsc.subcore_barrier()
        pltpu.sync_copy(accum.at[pl.ds(r0, rps)],
                        out_hbm.at[cid, pl.ds(r0, rps)])
        if tail:
            @pl.when(sid == 0)
            def _():
                pltpu.sync_copy(accum.at[pl.ds(rps * n_sub, tail)],
                                out_hbm.at[cid, pl.ds(rps * n_sub, tail)])

    return conv(xk_flat, fidx3, dst3, zeros)


def _norm_mm_body(tn, p_ref, g_ref, b_ref, w_ref, o_ref, h_ref):
    @pl.when((pl.program_id(0) == 0) & (pl.program_id(1) == 0))
    def _():
        h = p_ref[0] + p_ref[1]
        mu = jnp.mean(h, axis=0, keepdims=True)
        d = h - mu
        var = jnp.mean(d * d, axis=0, keepdims=True)
        y = d * lax.rsqrt(var + _EPS) * g_ref[...] + b_ref[...]
        h_ref[...] = jnp.where(y >= 0, y, _SLOPE * y)

    i = pl.program_id(0)
    o_ref[0] = jnp.dot(h_ref[pl.ds(i * tn, tn), :], w_ref[0],
                       preferred_element_type=jnp.float32)


def _norm_mm(p, gamma, beta, W, tn=2000):
    n, c = p.shape[1], p.shape[2]
    k = W.shape[0]
    return pl.pallas_call(
        functools.partial(_norm_mm_body, tn),
        grid=(n // tn, k),
        in_specs=[
            pl.BlockSpec((2, n, c), lambda i, j: (0, 0, 0)),
            pl.BlockSpec((1, c), lambda i, j: (0, 0)),
            pl.BlockSpec((1, c), lambda i, j: (0, 0)),
            pl.BlockSpec((1, c, c), lambda i, j: (j, 0, 0)),
        ],
        out_specs=pl.BlockSpec((1, tn, c), lambda i, j: (j, i, 0)),
        out_shape=jax.ShapeDtypeStruct((k, n, c), jnp.float32),
        scratch_shapes=[pltpu.VMEM((n, c), jnp.float32)],
    )(p, gamma.reshape(1, c), beta.reshape(1, c), W)


def _norm2_body(p_ref, g_ref, b_ref, r_ref, o_ref):
    h = p_ref[0] + p_ref[1]
    mu = jnp.mean(h, axis=0, keepdims=True)
    d = h - mu
    var = jnp.mean(d * d, axis=0, keepdims=True)
    y = d * lax.rsqrt(var + _EPS) * g_ref[...] + b_ref[...] + r_ref[...]
    o_ref[...] = jnp.where(y >= 0, y, _SLOPE * y)


def _norm2(p, gamma, beta, resid):
    n, c = p.shape[1], p.shape[2]
    return pl.pallas_call(
        _norm2_body,
        out_shape=jax.ShapeDtypeStruct((n, c), jnp.float32),
    )(p, gamma.reshape(1, c), beta.reshape(1, c), resid)


def kernel(x, edge_index, kernel_idx, W1, gamma1, beta1, W2, gamma2, beta2):
    n, c = x.shape
    k = W1.shape[0]
    src = edge_index[0]
    dst = edge_index[1]
    e = src.shape[0]
    nw, ch, nblk = 32, 125, 5
    blk = e // (nw * ch * nblk)
    fidx3 = _fidx(kernel_idx, src, n).reshape(nw * nblk, blk, ch)
    dst3 = dst.reshape(nw * nblk, blk, ch)
    zeros = jnp.zeros((n, c), jnp.float32)

    xk1 = _xk(x, W1)
    p1 = _sc_conv(xk1.reshape(k * n, c), fidx3, dst3, zeros, n, c, nblk)
    xk2 = _norm_mm(p1, gamma1, beta1, W2)
    p2 = _sc_conv(xk2.reshape(k * n, c), fidx3, dst3, zeros, n, c, nblk)
    return _norm2(p2, gamma2, beta2, x)


# speedup vs baseline: 3.7387x; 1.0821x over previous
import functools

import jax
import jax.numpy as jnp
from jax import lax
from jax.experimental import pallas as pl
from jax.experimental.pallas import tpu as pltpu
from jax.experimental.pallas import tpu_sc as plsc

_EPS = 1e-5
_SLOPE = 0.01


def _mm_body(x_ref, w_ref, o_ref):
    o_ref[0] = jnp.dot(x_ref[...], w_ref[0], preferred_element_type=jnp.float32)


def _xk(x, W, tn=2000):
    n, c = x.shape
    k = W.shape[0]
    return pl.pallas_call(
        _mm_body,
        grid=(n // tn, k),
        in_specs=[
            pl.BlockSpec((tn, c), lambda i, j: (i, 0)),
            pl.BlockSpec((1, c, c), lambda i, j: (j, 0, 0)),
        ],
        out_specs=pl.BlockSpec((1, tn, c), lambda i, j: (j, i, 0)),
        out_shape=jax.ShapeDtypeStruct((k, n, c), jnp.float32),
    )(x, W)


def _fidx_body(n, k_ref, s_ref, o_ref):
    o_ref[...] = k_ref[...] * n + s_ref[...]


def _fidx(kidx, src, n):
    e = kidx.shape[0]
    k2 = kidx.reshape(e // 128, 128)
    s2 = src.reshape(e // 128, 128)
    out = pl.pallas_call(
        functools.partial(_fidx_body, n),
        out_shape=jax.ShapeDtypeStruct(k2.shape, jnp.int32),
    )(k2, s2)
    return out.reshape(e)


def _sc_conv(xk_flat, fidx3, dst3, zeros, n, c, nblk):
    nwb, blk, ch = fidx3.shape
    n_cores, n_sub = 2, 16
    rps = (n // n_sub) // 8 * 8
    tail = n - rps * n_sub
    mesh = plsc.VectorSubcoreMesh(core_axis_name="c", subcore_axis_name="s")

    @functools.partial(
        pl.kernel,
        out_type=jax.ShapeDtypeStruct((n_cores, n, c), jnp.float32),
        mesh=mesh,
        scratch_types=[
            pltpu.VMEM((blk, ch), jnp.int32),
            pltpu.VMEM((blk, ch), jnp.int32),
            pltpu.VMEM((ch, c), jnp.float32),
            pltpu.VMEM((ch, c), jnp.float32),
            pltpu.VMEM_SHARED((n, c), jnp.float32),
            pltpu.SemaphoreType.DMA,
            pltpu.SemaphoreType.DMA,
        ],
    )
    def conv(xk_hbm, fidx_hbm, dst_hbm, z_hbm, out_hbm,
             fidx_v, dst_v, rows0, rows1, accum, gsem, ssem):
        cid = lax.axis_index("c")
        sid = lax.axis_index("s")
        wid = cid * n_sub + sid
        rows = (rows0, rows1)

        r0 = sid * rps
        pltpu.sync_copy(z_hbm.at[pl.ds(r0, rps)], accum.at[pl.ds(r0, rps)])
        if tail:
            @pl.when(sid == 0)
            def _():
                pltpu.sync_copy(z_hbm.at[pl.ds(rps * n_sub, tail)],
                                accum.at[pl.ds(rps * n_sub, tail)])
        plsc.subcore_barrier()

        def g_start(i, b):
            return pltpu.async_copy(xk_hbm.at[fidx_v.at[i]], rows[b], gsem)

        def s_start(i, b):
            return pltpu.async_copy(rows[b], accum.at[dst_v.at[i]],
                                    ssem, add=True)

        @pl.loop(0, nblk)
        def _(b):
            plane = wid * nblk + b
            pltpu.sync_copy(fidx_hbm.at[plane], fidx_v)
            pltpu.sync_copy(dst_hbm.at[plane], dst_v)

            g_start(0, 0).wait()

            @pl.loop(0, blk // 2 - 1)
            def _(q):
                p = 2 * q
                hs0 = s_start(p, 0)
                hg0 = g_start(p + 1, 1)
                hg0.wait()
                hs0.wait()
                hs1 = s_start(p + 1, 1)
                hg1 = g_start(p + 2, 0)
                hg1.wait()
                hs1.wait()

            hs0 = s_start(blk - 2, 0)
            hg0 = g_start(blk - 1, 1)
            hg0.wait()
            hs0.wait()
            s_start(blk - 1, 1).wait()

        plsc.subcore_barrier()
        pltpu.sync_copy(accum.at[pl.ds(r0, rps)],
                        out_hbm.at[cid, pl.ds(r0, rps)])
        if tail:
            @pl.when(sid == 0)
            def _():
                pltpu.sync_copy(accum.at[pl.ds(rps * n_sub, tail)],
                                out_hbm.at[cid, pl.ds(rps * n_sub, tail)])

    return conv(xk_flat, fidx3, dst3, zeros)


def _norm_mm_body(tn, p_ref, g_ref, b_ref, w_ref, o_ref, h_ref):
    @pl.when((pl.program_id(0) == 0) & (pl.program_id(1) == 0))
    def _():
        h = p_ref[0] + p_ref[1]
        mu = jnp.mean(h, axis=0, keepdims=True)
        d = h - mu
        var = jnp.mean(d * d, axis=0, keepdims=True)
        y = d * lax.rsqrt(var + _EPS) * g_ref[...] + b_ref[...]
        h_ref[...] = jnp.where(y >= 0, y, _SLOPE * y)

    i = pl.program_id(0)
    o_ref[0] = jnp.dot(h_ref[pl.ds(i * tn, tn), :], w_ref[0],
                       preferred_element_type=jnp.float32)


def _norm_mm(p, gamma, beta, W, tn=2000):
    n, c = p.shape[1], p.shape[2]
    k = W.shape[0]
    return pl.pallas_call(
        functools.partial(_norm_mm_body, tn),
        grid=(n // tn, k),
        in_specs=[
            pl.BlockSpec((2, n, c), lambda i, j: (0, 0, 0)),
            pl.BlockSpec((1, c), lambda i, j: (0, 0)),
            pl.BlockSpec((1, c), lambda i, j: (0, 0)),
            pl.BlockSpec((1, c, c), lambda i, j: (j, 0, 0)),
        ],
        out_specs=pl.BlockSpec((1, tn, c), lambda i, j: (j, i, 0)),
        out_shape=jax.ShapeDtypeStruct((k, n, c), jnp.float32),
        scratch_shapes=[pltpu.VMEM((n, c), jnp.float32)],
    )(p, gamma.reshape(1, c), beta.reshape(1, c), W)


def _norm2_body(p_ref, g_ref, b_ref, r_ref, o_ref):
    h = p_ref[0] + p_ref[1]
    mu = jnp.mean(h, axis=0, keepdims=True)
    d = h - mu
    var = jnp.mean(d * d, axis=0, keepdims=True)
    y = d * lax.rsqrt(var + _EPS) * g_ref[...] + b_ref[...] + r_ref[...]
    o_ref[...] = jnp.where(y >= 0, y, _SLOPE * y)


def _norm2(p, gamma, beta, resid):
    n, c = p.shape[1], p.shape[2]
    return pl.pallas_call(
        _norm2_body,
        out_shape=jax.ShapeDtypeStruct((n, c), jnp.float32),
    )(p, gamma.reshape(1, c), beta.reshape(1, c), resid)


def kernel(x, edge_index, kernel_idx, W1, gamma1, beta1, W2, gamma2, beta2):
    n, c = x.shape
    k = W1.shape[0]
    src = edge_index[0]
    dst = edge_index[1]
    e = src.shape[0]
    nw, ch, nblk = 32, 125, 2
    blk = e // (nw * ch * nblk)
    fidx3 = _fidx(kernel_idx, src, n).reshape(nw * nblk, blk, ch)
    dst3 = dst.reshape(nw * nblk, blk, ch)
    zeros = jnp.zeros((n, c), jnp.float32)

    xk1 = _xk(x, W1)
    p1 = _sc_conv(xk1.reshape(k * n, c), fidx3, dst3, zeros, n, c, nblk)
    xk2 = _norm_mm(p1, gamma1, beta1, W2)
    p2 = _sc_conv(xk2.reshape(k * n, c), fidx3, dst3, zeros, n, c, nblk)
    return _norm2(p2, gamma2, beta2, x)


# speedup vs baseline: 4.3798x; 1.1715x over previous
import functools

import jax
import jax.numpy as jnp
from jax import lax
from jax.experimental import pallas as pl
from jax.experimental.pallas import tpu as pltpu
from jax.experimental.pallas import tpu_sc as plsc

_EPS = 1e-5
_SLOPE = 0.01


def _mm_body(x_ref, w_ref, o_ref):
    o_ref[0] = jnp.dot(x_ref[...], w_ref[0], preferred_element_type=jnp.float32)


def _xk(x, W, tn=5000):
    n, c = x.shape
    k = W.shape[0]
    return pl.pallas_call(
        _mm_body,
        grid=(n // tn, k),
        in_specs=[
            pl.BlockSpec((tn, c), lambda i, j: (i, 0)),
            pl.BlockSpec((1, c, c), lambda i, j: (j, 0, 0)),
        ],
        out_specs=pl.BlockSpec((1, tn, c), lambda i, j: (j, i, 0)),
        out_shape=jax.ShapeDtypeStruct((k, n, c), jnp.float32),
    )(x, W)


def _fidx_body(n, k_ref, s_ref, o_ref):
    o_ref[...] = k_ref[...] * n + s_ref[...]


def _fidx(kidx, src, n):
    e = kidx.shape[0]
    k2 = kidx.reshape(e // 128, 128)
    s2 = src.reshape(e // 128, 128)
    out = pl.pallas_call(
        functools.partial(_fidx_body, n),
        out_shape=jax.ShapeDtypeStruct(k2.shape, jnp.int32),
    )(k2, s2)
    return out.reshape(e)


def _sc_conv(xk_flat, fidx3, dst3, zeros, n, c, nblk):
    nwb, blk, ch = fidx3.shape
    n_cores, n_sub = 2, 16
    rps = (n // n_sub) // 8 * 8
    tail = n - rps * n_sub
    mesh = plsc.VectorSubcoreMesh(core_axis_name="c", subcore_axis_name="s")

    @functools.partial(
        pl.kernel,
        out_type=jax.ShapeDtypeStruct((n_cores, n, c), jnp.float32),
        mesh=mesh,
        scratch_types=[
            pltpu.VMEM((blk, ch), jnp.int32),
            pltpu.VMEM((blk, ch), jnp.int32),
            pltpu.VMEM((ch, c), jnp.float32),
            pltpu.VMEM((ch, c), jnp.float32),
            pltpu.VMEM_SHARED((n, c), jnp.float32),
            pltpu.SemaphoreType.DMA,
            pltpu.SemaphoreType.DMA,
        ],
    )
    def conv(xk_hbm, fidx_hbm, dst_hbm, z_hbm, out_hbm,
             fidx_v, dst_v, rows0, rows1, accum, gsem, ssem):
        cid = lax.axis_index("c")
        sid = lax.axis_index("s")
        wid = cid * n_sub + sid
        rows = (rows0, rows1)

        r0 = sid * rps
        pltpu.sync_copy(z_hbm.at[pl.ds(r0, rps)], accum.at[pl.ds(r0, rps)])
        if tail:
            @pl.when(sid == 0)
            def _():
                pltpu.sync_copy(z_hbm.at[pl.ds(rps * n_sub, tail)],
                                accum.at[pl.ds(rps * n_sub, tail)])
        plsc.subcore_barrier()

        def g_start(i, b):
            return pltpu.async_copy(xk_hbm.at[fidx_v.at[i]], rows[b], gsem)

        def s_start(i, b):
            return pltpu.async_copy(rows[b], accum.at[dst_v.at[i]],
                                    ssem, add=True)

        @pl.loop(0, nblk)
        def _(b):
            plane = wid * nblk + b
            pltpu.sync_copy(fidx_hbm.at[plane], fidx_v)
            pltpu.sync_copy(dst_hbm.at[plane], dst_v)

            g_start(0, 0).wait()

            @pl.loop(0, blk // 2 - 1)
            def _(q):
                p = 2 * q
                hs0 = s_start(p, 0)
                hg0 = g_start(p + 1, 1)
                hg0.wait()
                hs0.wait()
                hs1 = s_start(p + 1, 1)
                hg1 = g_start(p + 2, 0)
                hg1.wait()
                hs1.wait()

            hs0 = s_start(blk - 2, 0)
            hg0 = g_start(blk - 1, 1)
            hg0.wait()
            hs0.wait()
            s_start(blk - 1, 1).wait()

        plsc.subcore_barrier()
        pltpu.sync_copy(accum.at[pl.ds(r0, rps)],
                        out_hbm.at[cid, pl.ds(r0, rps)])
        if tail:
            @pl.when(sid == 0)
            def _():
                pltpu.sync_copy(accum.at[pl.ds(rps * n_sub, tail)],
                                out_hbm.at[cid, pl.ds(rps * n_sub, tail)])

    return conv(xk_flat, fidx3, dst3, zeros)


def _norm_mm_body(tn, p_ref, g_ref, b_ref, w_ref, o_ref, h_ref):
    @pl.when((pl.program_id(0) == 0) & (pl.program_id(1) == 0))
    def _():
        h = p_ref[0] + p_ref[1]
        mu = jnp.mean(h, axis=0, keepdims=True)
        d = h - mu
        var = jnp.mean(d * d, axis=0, keepdims=True)
        y = d * lax.rsqrt(var + _EPS) * g_ref[...] + b_ref[...]
        h_ref[...] = jnp.where(y >= 0, y, _SLOPE * y)

    i = pl.program_id(0)
    o_ref[0] = jnp.dot(h_ref[pl.ds(i * tn, tn), :], w_ref[0],
                       preferred_element_type=jnp.float32)


def _norm_mm(p, gamma, beta, W, tn=5000):
    n, c = p.shape[1], p.shape[2]
    k = W.shape[0]
    return pl.pallas_call(
        functools.partial(_norm_mm_body, tn),
        grid=(n // tn, k),
        in_specs=[
            pl.BlockSpec((2, n, c), lambda i, j: (0, 0, 0)),
            pl.BlockSpec((1, c), lambda i, j: (0, 0)),
            pl.BlockSpec((1, c), lambda i, j: (0, 0)),
            pl.BlockSpec((1, c, c), lambda i, j: (j, 0, 0)),
        ],
        out_specs=pl.BlockSpec((1, tn, c), lambda i, j: (j, i, 0)),
        out_shape=jax.ShapeDtypeStruct((k, n, c), jnp.float32),
        scratch_shapes=[pltpu.VMEM((n, c), jnp.float32)],
    )(p, gamma.reshape(1, c), beta.reshape(1, c), W)


def _norm2_body(p_ref, g_ref, b_ref, r_ref, o_ref):
    h = p_ref[0] + p_ref[1]
    mu = jnp.mean(h, axis=0, keepdims=True)
    d = h - mu
    var = jnp.mean(d * d, axis=0, keepdims=True)
    y = d * lax.rsqrt(var + _EPS) * g_ref[...] + b_ref[...] + r_ref[...]
    o_ref[...] = jnp.where(y >= 0, y, _SLOPE * y)


def _norm2(p, gamma, beta, resid):
    n, c = p.shape[1], p.shape[2]
    return pl.pallas_call(
        _norm2_body,
        out_shape=jax.ShapeDtypeStruct((n, c), jnp.float32),
    )(p, gamma.reshape(1, c), beta.reshape(1, c), resid)


def kernel(x, edge_index, kernel_idx, W1, gamma1, beta1, W2, gamma2, beta2):
    n, c = x.shape
    k = W1.shape[0]
    src = edge_index[0]
    dst = edge_index[1]
    e = src.shape[0]
    nw, ch, nblk = 32, 125, 2
    blk = e // (nw * ch * nblk)
    fidx3 = _fidx(kernel_idx, src, n).reshape(nw * nblk, blk, ch)
    dst3 = dst.reshape(nw * nblk, blk, ch)
    zeros = jnp.zeros((n, c), jnp.float32)

    xk1 = _xk(x, W1)
    p1 = _sc_conv(xk1.reshape(k * n, c), fidx3, dst3, zeros, n, c, nblk)
    xk2 = _norm_mm(p1, gamma1, beta1, W2)
    p2 = _sc_conv(xk2.reshape(k * n, c), fidx3, dst3, zeros, n, c, nblk)
    return _norm2(p2, gamma2, beta2, x)


# speedup vs baseline: 4.8732x; 1.1127x over previous
import functools

import jax
import jax.numpy as jnp
from jax import lax
from jax.experimental import pallas as pl
from jax.experimental.pallas import tpu as pltpu
from jax.experimental.pallas import tpu_sc as plsc

_EPS = 1e-5
_SLOPE = 0.01


def _mm_body(x_ref, w_ref, o_ref):
    o_ref[0] = jnp.dot(x_ref[...], w_ref[0], preferred_element_type=jnp.float32)


def _xk(x, W, tn=10000):
    n, c = x.shape
    k = W.shape[0]
    return pl.pallas_call(
        _mm_body,
        grid=(n // tn, k),
        in_specs=[
            pl.BlockSpec((tn, c), lambda i, j: (i, 0)),
            pl.BlockSpec((1, c, c), lambda i, j: (j, 0, 0)),
        ],
        out_specs=pl.BlockSpec((1, tn, c), lambda i, j: (j, i, 0)),
        out_shape=jax.ShapeDtypeStruct((k, n, c), jnp.float32),
    )(x, W)


def _fidx_body(n, k_ref, s_ref, o_ref):
    o_ref[...] = k_ref[...] * n + s_ref[...]


def _fidx(kidx, src, n):
    e = kidx.shape[0]
    k2 = kidx.reshape(e // 128, 128)
    s2 = src.reshape(e // 128, 128)
    out = pl.pallas_call(
        functools.partial(_fidx_body, n),
        out_shape=jax.ShapeDtypeStruct(k2.shape, jnp.int32),
    )(k2, s2)
    return out.reshape(e)


def _sc_conv(xk_flat, fidx3, dst3, zeros, n, c, nblk):
    nwb, blk, ch = fidx3.shape
    n_cores, n_sub = 2, 16
    rps = (n // n_sub) // 8 * 8
    tail = n - rps * n_sub
    mesh = plsc.VectorSubcoreMesh(core_axis_name="c", subcore_axis_name="s")

    @functools.partial(
        pl.kernel,
        out_type=jax.ShapeDtypeStruct((n_cores, n, c), jnp.float32),
        mesh=mesh,
        scratch_types=[
            pltpu.VMEM((blk, ch), jnp.int32),
            pltpu.VMEM((blk, ch), jnp.int32),
            pltpu.VMEM((ch, c), jnp.float32),
            pltpu.VMEM((ch, c), jnp.float32),
            pltpu.VMEM_SHARED((n, c), jnp.float32),
            pltpu.SemaphoreType.DMA,
            pltpu.SemaphoreType.DMA,
        ],
    )
    def conv(xk_hbm, fidx_hbm, dst_hbm, z_hbm, out_hbm,
             fidx_v, dst_v, rows0, rows1, accum, gsem, ssem):
        cid = lax.axis_index("c")
        sid = lax.axis_index("s")
        wid = cid * n_sub + sid
        rows = (rows0, rows1)

        r0 = sid * rps
        pltpu.sync_copy(z_hbm.at[pl.ds(r0, rps)], accum.at[pl.ds(r0, rps)])
        if tail:
            @pl.when(sid == 0)
            def _():
                pltpu.sync_copy(z_hbm.at[pl.ds(rps * n_sub, tail)],
                                accum.at[pl.ds(rps * n_sub, tail)])
        plsc.subcore_barrier()

        def g_start(i, b):
            return pltpu.async_copy(xk_hbm.at[fidx_v.at[i]], rows[b], gsem)

        def s_start(i, b):
            return pltpu.async_copy(rows[b], accum.at[dst_v.at[i]],
                                    ssem, add=True)

        @pl.loop(0, nblk)
        def _(b):
            plane = wid * nblk + b
            pltpu.sync_copy(fidx_hbm.at[plane], fidx_v)
            pltpu.sync_copy(dst_hbm.at[plane], dst_v)

            g_start(0, 0).wait()

            @pl.loop(0, blk // 2 - 1)
            def _(q):
                p = 2 * q
                hs0 = s_start(p, 0)
                hg0 = g_start(p + 1, 1)
                hg0.wait()
                hs0.wait()
                hs1 = s_start(p + 1, 1)
                hg1 = g_start(p + 2, 0)
                hg1.wait()
                hs1.wait()

            hs0 = s_start(blk - 2, 0)
            hg0 = g_start(blk - 1, 1)
            hg0.wait()
            hs0.wait()
            s_start(blk - 1, 1).wait()

        plsc.subcore_barrier()
        pltpu.sync_copy(accum.at[pl.ds(r0, rps)],
                        out_hbm.at[cid, pl.ds(r0, rps)])
        if tail:
            @pl.when(sid == 0)
            def _():
                pltpu.sync_copy(accum.at[pl.ds(rps * n_sub, tail)],
                                out_hbm.at[cid, pl.ds(rps * n_sub, tail)])

    return conv(xk_flat, fidx3, dst3, zeros)


def _norm_mm_body(tn, p_ref, g_ref, b_ref, w_ref, o_ref, h_ref):
    @pl.when((pl.program_id(0) == 0) & (pl.program_id(1) == 0))
    def _():
        h = p_ref[0] + p_ref[1]
        mu = jnp.mean(h, axis=0, keepdims=True)
        d = h - mu
        var = jnp.mean(d * d, axis=0, keepdims=True)
        y = d * lax.rsqrt(var + _EPS) * g_ref[...] + b_ref[...]
        h_ref[...] = jnp.where(y >= 0, y, _SLOPE * y)

    i = pl.program_id(0)
    o_ref[0] = jnp.dot(h_ref[pl.ds(i * tn, tn), :], w_ref[0],
                       preferred_element_type=jnp.float32)


def _norm_mm(p, gamma, beta, W, tn=10000):
    n, c = p.shape[1], p.shape[2]
    k = W.shape[0]
    return pl.pallas_call(
        functools.partial(_norm_mm_body, tn),
        grid=(n // tn, k),
        in_specs=[
            pl.BlockSpec((2, n, c), lambda i, j: (0, 0, 0)),
            pl.BlockSpec((1, c), lambda i, j: (0, 0)),
            pl.BlockSpec((1, c), lambda i, j: (0, 0)),
            pl.BlockSpec((1, c, c), lambda i, j: (j, 0, 0)),
        ],
        out_specs=pl.BlockSpec((1, tn, c), lambda i, j: (j, i, 0)),
        out_shape=jax.ShapeDtypeStruct((k, n, c), jnp.float32),
        scratch_shapes=[pltpu.VMEM((n, c), jnp.float32)],
    )(p, gamma.reshape(1, c), beta.reshape(1, c), W)


def _norm2_body(p_ref, g_ref, b_ref, r_ref, o_ref):
    h = p_ref[0] + p_ref[1]
    mu = jnp.mean(h, axis=0, keepdims=True)
    d = h - mu
    var = jnp.mean(d * d, axis=0, keepdims=True)
    y = d * lax.rsqrt(var + _EPS) * g_ref[...] + b_ref[...] + r_ref[...]
    o_ref[...] = jnp.where(y >= 0, y, _SLOPE * y)


def _norm2(p, gamma, beta, resid):
    n, c = p.shape[1], p.shape[2]
    return pl.pallas_call(
        _norm2_body,
        out_shape=jax.ShapeDtypeStruct((n, c), jnp.float32),
    )(p, gamma.reshape(1, c), beta.reshape(1, c), resid)


def kernel(x, edge_index, kernel_idx, W1, gamma1, beta1, W2, gamma2, beta2):
    n, c = x.shape
    k = W1.shape[0]
    src = edge_index[0]
    dst = edge_index[1]
    e = src.shape[0]
    nw, ch, nblk = 32, 125, 2
    blk = e // (nw * ch * nblk)
    fidx3 = _fidx(kernel_idx, src, n).reshape(nw * nblk, blk, ch)
    dst3 = dst.reshape(nw * nblk, blk, ch)
    zeros = jnp.zeros((n, c), jnp.float32)

    xk1 = _xk(x, W1)
    p1 = _sc_conv(xk1.reshape(k * n, c), fidx3, dst3, zeros, n, c, nblk)
    xk2 = _norm_mm(p1, gamma1, beta1, W2)
    p2 = _sc_conv(xk2.reshape(k * n, c), fidx3, dst3, zeros, n, c, nblk)
    return _norm2(p2, gamma2, beta2, x)
